# baseline (speedup 1.0000x reference)
import functools

import jax
import jax.numpy as jnp
from jax import lax
from jax.experimental import pallas as pl
from jax.experimental.pallas import tpu as pltpu
from jax.experimental.pallas import tpu_sc as plsc

_B = 16
_H = 512
_W = 512
_HW = _H * _W
_N = 2000
_NC = 2
_NS = 16
_NWORK = _NC * _NS
_NPAD = 2048
_SPW = _NPAD // _NWORK
_CHUNKS = _SPW // 16
_GROWS = (_B * _SPW) // 128


def _station_loss_body(pred_hbm, tgt_hbm, rows_hbm, cols_hbm, out_hbm,
                       rows_v, cols_v, tgt_v, idx_v, vals_v, acc_v, sem):
    cid = lax.axis_index("c")
    sid = lax.axis_index("s")
    wid = sid * _NC + cid
    base = wid * _SPW

    pltpu.sync_copy(rows_hbm.at[pl.ds(base, _SPW)], rows_v)
    pltpu.sync_copy(cols_hbm.at[pl.ds(base, _SPW)], cols_v)
    for b in range(_B):
        pltpu.sync_copy(tgt_hbm.at[pl.ds(b * _NPAD + base, _SPW)], tgt_v.at[b])

    for j in range(_CHUNKS):
        r = rows_v[pl.ds(j * 16, 16)]
        c = cols_v[pl.ds(j * 16, 16)]
        flat = r * _W + c
        for b in range(_B):
            p = b * _SPW + j * 16
            idx_v[p // 128, pl.ds(p % 128, 16)] = flat + (b * _HW)

    copies = [
        pltpu.async_copy(pred_hbm.at[idx_v.at[g]], vals_v.at[g], sem)
        for g in range(_GROWS)
    ]
    for cp in copies:
        cp.wait()

    lane = lax.iota(jnp.int32, 16)
    masks = [(base + j * 16 + lane) < _N for j in range(_CHUNKS)]

    acc = jnp.zeros((16,), jnp.float32)
    for b in range(_B):
        for j in range(_CHUNKS):
            p = b * _SPW + j * 16
            v = vals_v[p // 128, pl.ds(p % 128, 16)]
            t = tgt_v[b, pl.ds(j * 16, 16)]
            d = jnp.abs(v - t)
            acc = acc + jnp.where(masks[j], d, 0.0)

    acc_v[...] = acc
    pltpu.sync_copy(acc_v, out_hbm.at[wid])


@functools.partial(jax.jit, static_argnames=())
def kernel(pred_images, target_runoff_values, station_rows, station_cols):
    pred_flat = pred_images.reshape(_B * _HW)
    tgt_t = jnp.zeros((_B, _NPAD), jnp.float32)
    tgt_t = tgt_t.at[:, :_N].set(target_runoff_values[:, :_B].T).reshape(-1)
    rows_p = jnp.zeros((_NPAD,), jnp.int32).at[:_N].set(station_rows)
    cols_p = jnp.zeros((_NPAD,), jnp.int32).at[:_N].set(station_cols)

    mesh = plsc.VectorSubcoreMesh(core_axis_name="c", subcore_axis_name="s")
    partials = pl.kernel(
        _station_loss_body,
        out_type=jax.ShapeDtypeStruct((_NWORK, 16), jnp.float32),
        mesh=mesh,
        scratch_types=[
            pltpu.VMEM((_SPW,), jnp.int32),
            pltpu.VMEM((_SPW,), jnp.int32),
            pltpu.VMEM((_B, _SPW), jnp.float32),
            pltpu.VMEM((_GROWS, 128), jnp.int32),
            pltpu.VMEM((_GROWS, 128), jnp.float32),
            pltpu.VMEM((16,), jnp.float32),
            pltpu.SemaphoreType.DMA,
        ],
    )(pred_flat, tgt_t, rows_p, cols_p)
    return jnp.sum(partials) / (_N * _B)


# speedup vs baseline: 1.3912x; 1.3912x over previous
import jax
import jax.numpy as jnp
from jax import lax
from jax.experimental import pallas as pl
from jax.experimental.pallas import tpu as pltpu
from jax.experimental.pallas import tpu_sc as plsc

_B = 16
_H = 512
_W = 512
_HW = _H * _W
_N = 2000
_NC = 2
_NS = 16
_NWORK = _NC * _NS
_NPAD = 2048
_ROWS_PER_W = (_B * _H) // _NWORK
_CH = 128
_NCHUNK = _ROWS_PER_W // _CH
_SENTINEL = 1 << 29


def _station_loss_body(pred_hbm, tgt_hbm, rows_hbm, cols_hbm, out_hbm,
                       rows_v, cols_v, tgt_v, vidx_v, slab_v, acc_v, sem):
    cid = lax.axis_index("c")
    sid = lax.axis_index("s")
    wid = sid * _NC + cid
    b = wid // 2
    row0 = wid * _ROWS_PER_W
    h0 = (wid % 2) * _ROWS_PER_W

    cp_r = pltpu.async_copy(rows_hbm, rows_v, sem)
    cp_c = pltpu.async_copy(cols_hbm, cols_v, sem)
    cp_t = pltpu.async_copy(tgt_hbm.at[pl.ds(b * _NPAD, _NPAD)], tgt_v, sem)
    cp_r.wait()
    cp_c.wait()
    cp_t.wait()

    lane = lax.iota(jnp.int32, 16)

    def pre_body(i, carry):
        base = i * 16
        r = rows_v[pl.ds(base, 16)]
        c = cols_v[pl.ds(base, 16)]
        rel = r - h0
        valid = (rel >= 0) & (rel < _ROWS_PER_W) & ((base + lane) < _N)
        vidx = jnp.where(valid, rel * _W + c, _SENTINEL)
        vidx_v[pl.ds(base, 16)] = vidx
        return carry

    lax.fori_loop(0, _NPAD // 16, pre_body, 0, unroll=4)

    acc = jnp.zeros((16,), jnp.float32)
    for k in range(_NCHUNK):
        pltpu.sync_copy(pred_hbm.at[pl.ds(row0 + k * _CH, _CH), :], slab_v)
        lo = k * _CH * _W

        def sweep_body(i, acc, lo=lo):
            base = i * 16
            v = vidx_v[pl.ds(base, 16)]
            m = (v >= lo) & (v < lo + _CH * _W)
            rel = jnp.where(m, v - lo, 0)
            x = plsc.load_gather(slab_v, [rel >> 9, rel & (_W - 1)], mask=m)
            t = tgt_v[pl.ds(base, 16)]
            d = jnp.abs(x - t)
            return acc + jnp.where(m, d, 0.0)

        acc = lax.fori_loop(0, _NPAD // 16, sweep_body, acc, unroll=4)

    acc_v[...] = acc
    pltpu.sync_copy(acc_v, out_hbm.at[wid])


def kernel(pred_images, target_runoff_values, station_rows, station_cols):
    pred2 = pred_images.reshape(_B * _H, _W)
    tgt_t = jnp.zeros((_B, _NPAD), jnp.float32)
    tgt_t = tgt_t.at[:, :_N].set(target_runoff_values[:, :_B].T).reshape(-1)
    rows_p = jnp.zeros((_NPAD,), jnp.int32).at[:_N].set(station_rows)
    cols_p = jnp.zeros((_NPAD,), jnp.int32).at[:_N].set(station_cols)

    mesh = plsc.VectorSubcoreMesh(core_axis_name="c", subcore_axis_name="s")
    partials = pl.kernel(
        _station_loss_body,
        out_type=jax.ShapeDtypeStruct((_NWORK, 16), jnp.float32),
        mesh=mesh,
        compiler_params=pltpu.CompilerParams(needs_layout_passes=False),
        scratch_types=[
            pltpu.VMEM((_NPAD,), jnp.int32),
            pltpu.VMEM((_NPAD,), jnp.int32),
            pltpu.VMEM((_NPAD,), jnp.float32),
            pltpu.VMEM((_NPAD,), jnp.int32),
            pltpu.VMEM((_CH, _W), jnp.float32),
            pltpu.VMEM((16,), jnp.float32),
            pltpu.SemaphoreType.DMA,
        ],
    )(pred2, tgt_t, rows_p, cols_p)
    return jnp.sum(partials) / (_N * _B)


# speedup vs baseline: 1.4487x; 1.0413x over previous
import jax
import jax.numpy as jnp
from jax import lax
from jax.experimental import pallas as pl
from jax.experimental.pallas import tpu as pltpu
from jax.experimental.pallas import tpu_sc as plsc

_B = 16
_H = 512
_W = 512
_N = 2000
_NWORK = 32
_ROWS_PER_W = (_B * _H) // _NWORK
_CH = 64
_NCHUNK = _ROWS_PER_W // _CH
_NVEC = _N // 16


def _station_loss_body(pred_hbm, tgt_hbm, rows_hbm, cols_hbm, out_hbm,
                       rows_v, cols_v, rl_v, tgtb_v, slab_a, slab_b, acc_v,
                       sem, sem_a, sem_b):
    cid = lax.axis_index("c")
    sid = lax.axis_index("s")
    wid = sid * 2 + cid
    b = wid // 2
    row0 = wid * _ROWS_PER_W
    h0 = (wid % 2) * _ROWS_PER_W

    slabs = [slab_a, slab_b]
    sems = [sem_a, sem_b]
    copies = [None] * _NCHUNK
    copies[0] = pltpu.async_copy(
        pred_hbm.at[pl.ds(row0, _CH), :], slab_a, sem_a)
    cp_r = pltpu.async_copy(rows_hbm, rows_v, sem)
    cp_c = pltpu.async_copy(cols_hbm, cols_v, sem)
    cp_t = pltpu.async_copy(tgt_hbm.at[pl.ds(b * _N, _N)], tgtb_v, sem)
    cp_r.wait()
    cp_c.wait()
    cp_t.wait()

    def pre_body(i, carry):
        base = i * 16
        r = rows_v[pl.ds(base, 16)]
        rel = r - h0
        rows_v[pl.ds(base, 16)] = rel >> 6
        rl_v[pl.ds(base, 16)] = rel & (_CH - 1)
        return carry

    lax.fori_loop(0, _NVEC, pre_body, 0, unroll=4)

    acc = jnp.zeros((16,), jnp.float32)
    for k in range(_NCHUNK):
        if k + 1 < _NCHUNK:
            copies[k + 1] = pltpu.async_copy(
                pred_hbm.at[pl.ds(row0 + (k + 1) * _CH, _CH), :],
                slabs[(k + 1) % 2], sems[(k + 1) % 2])
        copies[k].wait()
        slab_v = slabs[k % 2]

        def sweep_body(i, acc, k=k, slab_v=slab_v):
            base = i * 16
            m = rows_v[pl.ds(base, 16)] == k
            x = plsc.load_gather(
                slab_v, [rl_v[pl.ds(base, 16)], cols_v[pl.ds(base, 16)]],
                mask=m)
            d = jnp.abs(x - tgtb_v[pl.ds(base, 16)])
            return acc + jnp.where(m, d, 0.0)

        acc = lax.fori_loop(0, _NVEC, sweep_body, acc, unroll=4)

    acc_v[...] = acc
    pltpu.sync_copy(acc_v, out_hbm.at[wid])


def kernel(pred_images, target_runoff_values, station_rows, station_cols):
    pred2 = pred_images.reshape(_B * _H, _W)
    tgt_t = target_runoff_values[:, :_B].T.reshape(-1)

    mesh = plsc.VectorSubcoreMesh(core_axis_name="c", subcore_axis_name="s")
    partials = pl.kernel(
        _station_loss_body,
        out_type=jax.ShapeDtypeStruct((_NWORK, 16), jnp.float32),
        mesh=mesh,
        compiler_params=pltpu.CompilerParams(needs_layout_passes=False),
        scratch_types=[
            pltpu.VMEM((_N,), jnp.int32),
            pltpu.VMEM((_N,), jnp.int32),
            pltpu.VMEM((_N,), jnp.int32),
            pltpu.VMEM((_N,), jnp.float32),
            pltpu.VMEM((_CH, _W), jnp.float32),
            pltpu.VMEM((_CH, _W), jnp.float32),
            pltpu.VMEM((16,), jnp.float32),
            pltpu.SemaphoreType.DMA,
            pltpu.SemaphoreType.DMA,
            pltpu.SemaphoreType.DMA,
        ],
    )(pred2, tgt_t, station_rows, station_cols)
    return jnp.sum(partials) / (_N * _B)


# speedup vs baseline: 1.4961x; 1.0327x over previous
import jax
import jax.numpy as jnp
from jax import lax
from jax.experimental import pallas as pl
from jax.experimental.pallas import tpu as pltpu
from jax.experimental.pallas import tpu_sc as plsc

_B = 16
_H = 512
_W = 512
_N = 2000
_NWORK = 32
_ROWS_PER_W = (_B * _H) // _NWORK
_CH = 128
_NCHUNK = _ROWS_PER_W // _CH
_NVEC = _N // 16


def _station_loss_body(pred_hbm, tgt_hbm, rows_hbm, cols_hbm, out_hbm,
                       rows_v, cols_v, tgtb_v, slab_v, acc_v,
                       sem, slab_sem):
    cid = lax.axis_index("c")
    sid = lax.axis_index("s")
    wid = sid * 2 + cid
    b = wid // 2
    row0 = wid * _ROWS_PER_W
    h0 = (wid % 2) * _ROWS_PER_W

    cp_slab0 = pltpu.async_copy(
        pred_hbm.at[pl.ds(row0, _CH), :], slab_v, slab_sem)
    cp_r = pltpu.async_copy(rows_hbm, rows_v, sem)
    cp_c = pltpu.async_copy(cols_hbm, cols_v, sem)
    cp_t = pltpu.async_copy(tgt_hbm.at[pl.ds(b * _N, _N)], tgtb_v, sem)
    cp_r.wait()
    cp_c.wait()
    cp_t.wait()

    def pre_body(i, carry):
        base = i * 16
        r = rows_v[pl.ds(base, 16)]
        c = cols_v[pl.ds(base, 16)]
        rows_v[pl.ds(base, 16)] = ((r - (h0 - _ROWS_PER_W)) << 9) | c
        return carry

    lax.fori_loop(0, _NVEC, pre_body, 0, unroll=4)
    cp_slab0.wait()

    acc = jnp.zeros((16,), jnp.float32)
    for k in range(_NCHUNK):
        if k > 0:
            pltpu.sync_copy(pred_hbm.at[pl.ds(row0 + k * _CH, _CH), :],
                            slab_v)

        def sweep_body(i, acc, k=k):
            base = i * 16
            pk = rows_v[pl.ds(base, 16)]
            m = (pk >> 16) == (_ROWS_PER_W >> 7) + k
            x = plsc.load_gather(
                slab_v, [(pk >> 9) & (_CH - 1), pk & (_W - 1)], mask=m)
            d = jnp.abs(x - tgtb_v[pl.ds(base, 16)])
            return acc + jnp.where(m, d, 0.0)

        acc = lax.fori_loop(0, _NVEC, sweep_body, acc, unroll=8)

    acc_v[...] = acc
    pltpu.sync_copy(acc_v, out_hbm.at[wid])


def kernel(pred_images, target_runoff_values, station_rows, station_cols):
    pred2 = pred_images.reshape(_B * _H, _W)
    tgt_t = target_runoff_values[:, :_B].T.reshape(-1)

    mesh = plsc.VectorSubcoreMesh(core_axis_name="c", subcore_axis_name="s")
    partials = pl.kernel(
        _station_loss_body,
        out_type=jax.ShapeDtypeStruct((_NWORK, 16), jnp.float32),
        mesh=mesh,
        compiler_params=pltpu.CompilerParams(needs_layout_passes=False),
        scratch_types=[
            pltpu.VMEM((_N,), jnp.int32),
            pltpu.VMEM((_N,), jnp.int32),
            pltpu.VMEM((_N,), jnp.float32),
            pltpu.VMEM((_CH, _W), jnp.float32),
            pltpu.VMEM((16,), jnp.float32),
            pltpu.SemaphoreType.DMA,
            pltpu.SemaphoreType.DMA,
        ],
    )(pred2, tgt_t, station_rows, station_cols)
    return jnp.sum(partials) / (_N * _B)
